# speedup vs baseline: 1.5071x; 1.5071x over previous
import functools

import jax
import jax.numpy as jnp
from jax import lax
from jax.experimental import pallas as pl
from jax.experimental.pallas import tpu as pltpu
from jax.experimental.pallas import tpu_sc as plsc

_VOCAB = 50265
_D = 768
_B = 4
_S = 4096
_N = _B * _S
_NC = 2
_NS = 16
_NW = _NC * _NS
_ROWS_PER_W = _N // _NW
_CHUNK = 64
_NCHUNKS = _ROWS_PER_W // _CHUNK
_LANES = 16
_VECS_PER_ROW = _D // _LANES


def _make_sc_kernel():
    mesh = plsc.VectorSubcoreMesh(core_axis_name="c", subcore_axis_name="s")

    @functools.partial(
        pl.kernel,
        out_type=jax.ShapeDtypeStruct((_N, _D), jnp.float32),
        mesh=mesh,
        scratch_types=[
            pltpu.VMEM((_ROWS_PER_W,), jnp.int32),
            pltpu.VMEM((_CHUNK, _D), jnp.float32),
            pltpu.VMEM((_CHUNK, _D), jnp.float32),
            pltpu.SemaphoreType.DMA,
        ],
    )
    def body(ids_hbm, word_hbm, pos_hbm, out_hbm, idx_v, rows_v, pos_v, sem):
        wid = lax.axis_index("s") * _NC + lax.axis_index("c")
        base = wid * _ROWS_PER_W
        pos_base = lax.rem(base, _S)
        pltpu.sync_copy(ids_hbm.at[pl.ds(base, _ROWS_PER_W)], idx_v)

        def chunk_step(c, carry):
            off = c * _CHUNK
            gather = pltpu.async_copy(
                word_hbm.at[idx_v.at[pl.ds(off, _CHUNK)]], rows_v, sem
            )
            pltpu.sync_copy(
                pos_hbm.at[pl.ds(pos_base + off, _CHUNK)], pos_v
            )
            gather.wait()

            def row_step(r, carry2):
                for k in range(_VECS_PER_ROW):
                    plsc.addupdate(
                        rows_v.at[r, pl.ds(k * _LANES, _LANES)],
                        pos_v[r, pl.ds(k * _LANES, _LANES)],
                    )
                return carry2

            lax.fori_loop(0, _CHUNK, row_step, 0, unroll=False)
            pltpu.sync_copy(rows_v, out_hbm.at[pl.ds(base + off, _CHUNK)])
            return carry

        lax.fori_loop(0, _NCHUNKS, chunk_step, 0, unroll=False)

    return body


_sc_kernel = _make_sc_kernel()


@jax.jit
def kernel(input_ids, word_embeddings, position_embeddings):
    ids_flat = jnp.reshape(input_ids.astype(jnp.int32), (_N,))
    out = _sc_kernel(ids_flat, word_embeddings, position_embeddings)
    return jnp.reshape(out, (_B, _S, _D))
